# Initial kernel scaffold; baseline (speedup 1.0000x reference)
import jax
import jax.numpy as jnp
from jax.experimental import pallas as pl


def kernel(x, edge_index, edge_attr, Wb1, Wc1, bc1, b1, Wb2, Wc2, bc2, b2, Wb3, Wc3, bc3, b3, Wb4, Wc4, bc4, b4):
    raise NotImplementedError("write your pallas kernel here")


# speedup vs baseline: 12.1918x; 12.1918x over previous
import functools

import numpy as np
import jax
import jax.numpy as jnp
from jax import lax
from jax.experimental import pallas as pl
from jax.experimental.pallas import tpu as pltpu
from jax.experimental.pallas import tpu_sc as plsc

_N = 10000
_NPAD = 10240
_E = 320000
_NW = 32
_CH = 128
_NCHUNK = 80
_EP = _NW * _NCHUNK * _CH
_D = 64
_DW = 32
_RPT = _NPAD // 16
_BN = 512
_GRID = _NPAD // _BN
_HEADS, _NB, _HD = 8, 4, 16


@functools.lru_cache(maxsize=None)
def _sc_deg():
    mesh = plsc.VectorSubcoreMesh(core_axis_name="c", subcore_axis_name="s")

    @functools.partial(
        pl.kernel, mesh=mesh,
        compiler_params=pltpu.CompilerParams(use_tc_tiling_on_sc=False),
        out_type=jax.ShapeDtypeStruct((2 * _NPAD, 16), jnp.float32),
        scratch_types=[
            pltpu.VMEM((_NCHUNK, _CH), jnp.int32),
            pltpu.VMEM((_CH, 16), jnp.float32),
            pltpu.VMEM_SHARED((_NPAD, 16), jnp.float32),
        ],
    )
    def deg_kernel(coli, ones, z16, out, colv, onesv, acc):
        c = lax.axis_index("c")
        s = lax.axis_index("s")
        wid = s * 2 + c
        pltpu.sync_copy(coli.at[wid], colv)
        pltpu.sync_copy(ones, onesv)
        pltpu.sync_copy(z16, acc.at[pl.ds(s * _RPT, _RPT)])
        plsc.subcore_barrier()

        def body(j, carry):
            pltpu.sync_copy(onesv, acc.at[colv.at[j]], add=True)
            return carry

        lax.fori_loop(0, _NCHUNK, body, 0)
        plsc.subcore_barrier()
        pltpu.sync_copy(acc.at[pl.ds(s * _RPT, _RPT)],
                        out.at[pl.ds(c * _NPAD + s * _RPT, _RPT)])

    return deg_kernel


@functools.lru_cache(maxsize=None)
def _sc_gather_scatter():
    mesh = plsc.VectorSubcoreMesh(core_axis_name="c", subcore_axis_name="s")

    @functools.partial(
        pl.kernel, mesh=mesh,
        compiler_params=pltpu.CompilerParams(use_tc_tiling_on_sc=False),
        out_type=jax.ShapeDtypeStruct((2 * _NPAD, _D), jnp.float32),
        scratch_types=[
            pltpu.VMEM((_NCHUNK, _CH), jnp.int32),
            pltpu.VMEM((_NCHUNK, _CH), jnp.int32),
            pltpu.VMEM((_CH, _D), jnp.float32),
            pltpu.VMEM((_CH, _D), jnp.float32),
            pltpu.VMEM_SHARED((_NPAD, _D), jnp.float32),
            pltpu.SemaphoreType.DMA,
            pltpu.SemaphoreType.DMA,
        ],
    )
    def gs_kernel(table, rowi, coli, z64, out,
                  rowv, colv, bufa, bufb, acc, sema, semb):
        c = lax.axis_index("c")
        s = lax.axis_index("s")
        wid = s * 2 + c
        pltpu.sync_copy(rowi.at[wid], rowv)
        pltpu.sync_copy(coli.at[wid], colv)
        pltpu.sync_copy(z64, acc.at[pl.ds(s * _RPT, _RPT)])
        plsc.subcore_barrier()

        pltpu.async_copy(table.at[rowv.at[0]], bufa, sema)

        def body(jj, carry):
            j0 = 2 * jj
            j1 = j0 + 1
            h1 = pltpu.async_copy(table.at[rowv.at[j1]], bufb, semb)
            pltpu.make_async_copy(table.at[rowv.at[j0]], bufa, sema).wait()
            pltpu.sync_copy(bufa, acc.at[colv.at[j0]], add=True)

            @pl.when(jj < _NCHUNK // 2 - 1)
            def _():
                pltpu.async_copy(table.at[rowv.at[j0 + 2]], bufa, sema)

            h1.wait()
            pltpu.sync_copy(bufb, acc.at[colv.at[j1]], add=True)
            return carry

        lax.fori_loop(0, _NCHUNK // 2, body, 0)
        plsc.subcore_barrier()
        pltpu.sync_copy(acc.at[pl.ds(s * _RPT, _RPT)],
                        out.at[pl.ds(c * _NPAD + s * _RPT, _RPT)])

    return gs_kernel


def _expm1(v):
    u = jnp.exp(v)
    return jnp.where(u == 1.0, v,
                     jnp.where(u > 0.0, (u - 1.0) * v / jnp.log(u), -1.0))


def _elu1(v):
    return jnp.where(v > 0, v, _expm1(v))


def _combine(w, agg, bias):
    r8 = lax.broadcasted_iota(jnp.int32, (_HEADS, 128), 0)
    c8 = lax.broadcasted_iota(jnp.int32, (_HEADS, 128), 1)
    e8 = (c8 // _HD == r8).astype(jnp.float32)
    r16 = lax.broadcasted_iota(jnp.int32, (_HD, 128), 0)
    c16 = lax.broadcasted_iota(jnp.int32, (_HD, 128), 1)
    e16 = (c16 % _HD == r16).astype(jnp.float32)
    out = bias
    for b in range(_NB):
        rep = jnp.dot(w[:, b * _HEADS:(b + 1) * _HEADS], e8,
                      preferred_element_type=jnp.float32)
        til = jnp.dot(agg[:, b * _HD:(b + 1) * _HD], e16,
                      preferred_element_type=jnp.float32)
        out = out + rep * til
    return out


def _dinv_of(d0, d1):
    return lax.rsqrt(1.0 + d0[...][:, :1] + d1[...][:, :1])


def _tc_first_body(d0, d1, x, wb, wc, bc, ob, ow):
    dinv = _dinv_of(d0, d1)
    xb = x[...]
    ob[...] = dinv * jnp.dot(xb, wb[...], preferred_element_type=jnp.float32)
    ow[...] = jnp.dot(xb, wc[...], preferred_element_type=jnp.float32) + bc[...]


def _tc_mid_body(s0, s1, bd, w, d0, d1, bias, wb_n, wc_n, bc_n, ob, ow):
    dinv = _dinv_of(d0, d1)
    agg = dinv * (s0[...] + s1[...] + bd[...])
    h = _elu1(_combine(w[...], agg, bias[...]))
    ob[...] = dinv * jnp.dot(h, wb_n[...], preferred_element_type=jnp.float32)
    ow[...] = jnp.dot(h, wc_n[...], preferred_element_type=jnp.float32) + bc_n[...]


def _tc_last_body(s0, s1, bd, w, d0, d1, bias, o):
    dinv = _dinv_of(d0, d1)
    agg = dinv * (s0[...] + s1[...] + bd[...])
    v = _combine(w[...], agg, bias[...])
    o[...] = jnp.where(v > 0, v, 128.0 * _expm1(v))


def _bn(width):
    return pl.BlockSpec((_BN, width), lambda i: (i, 0))


def _full(rows, cols):
    return pl.BlockSpec((rows, cols), lambda i: (0, 0))


def _tc_first(d0, d1, xp, wb, wc, bc):
    return pl.pallas_call(
        _tc_first_body,
        grid=(_GRID,),
        in_specs=[_bn(16), _bn(16), _bn(128),
                  _full(128, _D), _full(128, _DW), _full(1, _DW)],
        out_specs=[_bn(_D), _bn(_DW)],
        out_shape=[jax.ShapeDtypeStruct((_NPAD, _D), jnp.float32),
                   jax.ShapeDtypeStruct((_NPAD, _DW), jnp.float32)],
    )(d0, d1, xp, wb, wc, bc)


def _tc_mid(s0, s1, bd, w, d0, d1, bias, wb_n, wc_n, bc_n):
    return pl.pallas_call(
        _tc_mid_body,
        grid=(_GRID,),
        in_specs=[_bn(_D), _bn(_D), _bn(_D), _bn(_DW), _bn(16), _bn(16),
                  _full(1, 128), _full(128, _D), _full(128, _DW), _full(1, _DW)],
        out_specs=[_bn(_D), _bn(_DW)],
        out_shape=[jax.ShapeDtypeStruct((_NPAD, _D), jnp.float32),
                   jax.ShapeDtypeStruct((_NPAD, _DW), jnp.float32)],
    )(s0, s1, bd, w, d0, d1, bias, wb_n, wc_n, bc_n)


def _tc_last(s0, s1, bd, w, d0, d1, bias):
    return pl.pallas_call(
        _tc_last_body,
        grid=(_GRID,),
        in_specs=[_bn(_D), _bn(_D), _bn(_D), _bn(_DW), _bn(16), _bn(16),
                  _full(1, 128)],
        out_specs=_bn(128),
        out_shape=jax.ShapeDtypeStruct((_NPAD, 128), jnp.float32),
    )(s0, s1, bd, w, d0, d1, bias)


_PERM = np.array([(j % _HEADS) * _NB + j // _HEADS for j in range(_DW)])


def kernel(x, edge_index, edge_attr,
           Wb1, Wc1, bc1, b1, Wb2, Wc2, bc2, b2,
           Wb3, Wc3, bc3, b3, Wb4, Wc4, bc4, b4):
    f32 = jnp.float32
    xp = jnp.pad(x, ((0, _NPAD - _N), (0, 0)))
    pad = _EP - _E
    row = jnp.concatenate(
        [edge_index[0], jnp.zeros((pad,), jnp.int32)]).reshape(_NW, _NCHUNK, _CH)
    col = jnp.concatenate(
        [edge_index[1], jnp.full((pad,), _N, jnp.int32)]).reshape(_NW, _NCHUNK, _CH)
    ones16 = jnp.ones((_CH, 16), f32)
    z16 = jnp.zeros((_RPT, 16), f32)
    z64 = jnp.zeros((_RPT, _D), f32)

    wbs = [Wb1, Wb2, Wb3, Wb4]
    wcs = [Wc1[:, _PERM], Wc2[:, _PERM], Wc3[:, _PERM], Wc4[:, _PERM]]
    bcs = [bc1[_PERM].reshape(1, _DW), bc2[_PERM].reshape(1, _DW),
           bc3[_PERM].reshape(1, _DW), bc4[_PERM].reshape(1, _DW)]
    bs = [b1.reshape(1, 128), b2.reshape(1, 128),
          b3.reshape(1, 128), b4.reshape(1, 128)]

    degp = _sc_deg()(col, ones16, z16)
    d0, d1 = degp[:_NPAD], degp[_NPAD:]

    gs = _sc_gather_scatter()
    bd, w = _tc_first(d0, d1, xp, wbs[0], wcs[0], bcs[0])
    for l in range(3):
        s = gs(bd, row, col, z64)
        bd, w = _tc_mid(s[:_NPAD], s[_NPAD:], bd, w, d0, d1,
                        bs[l], wbs[l + 1], wcs[l + 1], bcs[l + 1])
    s = gs(bd, row, col, z64)
    out = _tc_last(s[:_NPAD], s[_NPAD:], bd, w, d0, d1, bs[3])
    return out[:_N]


# speedup vs baseline: 12.5116x; 1.0262x over previous
import functools

import numpy as np
import jax
import jax.numpy as jnp
from jax import lax
from jax.experimental import pallas as pl
from jax.experimental.pallas import tpu as pltpu
from jax.experimental.pallas import tpu_sc as plsc

_N = 10000
_NPAD = 10240
_E = 320000
_NW = 32
_CH = 128
_NCHUNK = 80
_EP = _NW * _NCHUNK * _CH
_D = 64
_DW = 32
_RPT = _NPAD // 16
_BN = 512
_GRID = _NPAD // _BN
_HEADS, _NB, _HD = 8, 4, 16
_NBUF = 4


@functools.lru_cache(maxsize=None)
def _sc_deg():
    mesh = plsc.VectorSubcoreMesh(core_axis_name="c", subcore_axis_name="s")

    @functools.partial(
        pl.kernel, mesh=mesh,
        compiler_params=pltpu.CompilerParams(use_tc_tiling_on_sc=False),
        out_type=jax.ShapeDtypeStruct((2 * _NPAD, 16), jnp.float32),
        scratch_types=[
            pltpu.VMEM((_NCHUNK, _CH), jnp.int32),
            pltpu.VMEM((_CH, 16), jnp.float32),
            pltpu.VMEM_SHARED((_NPAD, 16), jnp.float32),
        ],
    )
    def deg_kernel(coli, ones, z16, out, colv, onesv, acc):
        c = lax.axis_index("c")
        s = lax.axis_index("s")
        wid = s * 2 + c
        pltpu.sync_copy(coli.at[wid], colv)
        pltpu.sync_copy(ones, onesv)
        pltpu.sync_copy(z16, acc.at[pl.ds(s * _RPT, _RPT)])
        plsc.subcore_barrier()

        def body(j, carry):
            pltpu.sync_copy(onesv, acc.at[colv.at[j]], add=True)
            return carry

        lax.fori_loop(0, _NCHUNK, body, 0)
        plsc.subcore_barrier()
        pltpu.sync_copy(acc.at[pl.ds(s * _RPT, _RPT)],
                        out.at[pl.ds(c * _NPAD + s * _RPT, _RPT)])

    return deg_kernel


@functools.lru_cache(maxsize=None)
def _sc_gather_scatter():
    mesh = plsc.VectorSubcoreMesh(core_axis_name="c", subcore_axis_name="s")

    @functools.partial(
        pl.kernel, mesh=mesh,
        compiler_params=pltpu.CompilerParams(use_tc_tiling_on_sc=False),
        out_type=jax.ShapeDtypeStruct((2 * _NPAD, _D), jnp.float32),
        scratch_types=[
            pltpu.VMEM((_NCHUNK, _CH), jnp.int32),
            pltpu.VMEM((_NCHUNK, _CH), jnp.int32),
            [pltpu.VMEM((_CH, _D), jnp.float32) for _ in range(_NBUF)],
            pltpu.VMEM_SHARED((_NPAD, _D), jnp.float32),
            [pltpu.SemaphoreType.DMA for _ in range(_NBUF)],
            [pltpu.SemaphoreType.DMA for _ in range(_NBUF)],
        ],
    )
    def gs_kernel(table, rowi, coli, z64, out,
                  rowv, colv, bufs, acc, gsems, ssems):
        c = lax.axis_index("c")
        s = lax.axis_index("s")
        wid = s * 2 + c
        pltpu.sync_copy(rowi.at[wid], rowv)
        pltpu.sync_copy(coli.at[wid], colv)
        pltpu.sync_copy(z64, acc.at[pl.ds(s * _RPT, _RPT)])
        plsc.subcore_barrier()

        for b in range(_NBUF):
            pltpu.async_copy(table.at[rowv.at[b]], bufs[b], gsems[b])

        def body(jj, carry):
            j = _NBUF * jj
            for b in range(_NBUF):
                pltpu.make_async_copy(
                    table.at[rowv.at[j + b]], bufs[b], gsems[b]).wait()
                pltpu.async_copy(bufs[b], acc.at[colv.at[j + b]], ssems[b],
                                 add=True)
            @pl.when(jj < _NCHUNK // _NBUF - 1)
            def _():
                for b in range(_NBUF):
                    pltpu.make_async_copy(
                        bufs[b], acc.at[colv.at[j + b]], ssems[b]).wait()
                    pltpu.async_copy(
                        table.at[rowv.at[j + _NBUF + b]], bufs[b], gsems[b])
            return carry

        lax.fori_loop(0, _NCHUNK // _NBUF, body, 0)
        j_last = _NCHUNK - _NBUF
        for b in range(_NBUF):
            pltpu.make_async_copy(
                bufs[b], acc.at[colv.at[j_last + b]], ssems[b]).wait()
        plsc.subcore_barrier()
        pltpu.sync_copy(acc.at[pl.ds(s * _RPT, _RPT)],
                        out.at[pl.ds(c * _NPAD + s * _RPT, _RPT)])

    return gs_kernel


def _expm1(v):
    u = jnp.exp(v)
    return jnp.where(u == 1.0, v,
                     jnp.where(u > 0.0, (u - 1.0) * v / jnp.log(u), -1.0))


def _elu1(v):
    return jnp.where(v > 0, v, _expm1(v))


def _combine(w, agg, bias):
    r8 = lax.broadcasted_iota(jnp.int32, (_HEADS, 128), 0)
    c8 = lax.broadcasted_iota(jnp.int32, (_HEADS, 128), 1)
    e8 = (c8 // _HD == r8).astype(jnp.float32)
    r16 = lax.broadcasted_iota(jnp.int32, (_HD, 128), 0)
    c16 = lax.broadcasted_iota(jnp.int32, (_HD, 128), 1)
    e16 = (c16 % _HD == r16).astype(jnp.float32)
    out = bias
    for b in range(_NB):
        rep = jnp.dot(w[:, b * _HEADS:(b + 1) * _HEADS], e8,
                      preferred_element_type=jnp.float32)
        til = jnp.dot(agg[:, b * _HD:(b + 1) * _HD], e16,
                      preferred_element_type=jnp.float32)
        out = out + rep * til
    return out


def _dinv_of(d0, d1):
    return lax.rsqrt(1.0 + d0[...][:, :1] + d1[...][:, :1])


def _tc_first_body(d0, d1, x, wb, wc, bc, ob, ow):
    dinv = _dinv_of(d0, d1)
    xb = x[...]
    ob[...] = dinv * jnp.dot(xb, wb[...], preferred_element_type=jnp.float32)
    ow[...] = jnp.dot(xb, wc[...], preferred_element_type=jnp.float32) + bc[...]


def _tc_mid_body(s0, s1, bd, w, d0, d1, bias, wb_n, wc_n, bc_n, ob, ow):
    dinv = _dinv_of(d0, d1)
    agg = dinv * (s0[...] + s1[...] + bd[...])
    h = _elu1(_combine(w[...], agg, bias[...]))
    ob[...] = dinv * jnp.dot(h, wb_n[...], preferred_element_type=jnp.float32)
    ow[...] = jnp.dot(h, wc_n[...], preferred_element_type=jnp.float32) + bc_n[...]


def _tc_last_body(s0, s1, bd, w, d0, d1, bias, o):
    dinv = _dinv_of(d0, d1)
    agg = dinv * (s0[...] + s1[...] + bd[...])
    v = _combine(w[...], agg, bias[...])
    o[...] = jnp.where(v > 0, v, 128.0 * _expm1(v))


def _bn(width):
    return pl.BlockSpec((_BN, width), lambda i: (i, 0))


def _full(rows, cols):
    return pl.BlockSpec((rows, cols), lambda i: (0, 0))


def _tc_first(d0, d1, xp, wb, wc, bc):
    return pl.pallas_call(
        _tc_first_body,
        grid=(_GRID,),
        in_specs=[_bn(16), _bn(16), _bn(128),
                  _full(128, _D), _full(128, _DW), _full(1, _DW)],
        out_specs=[_bn(_D), _bn(_DW)],
        out_shape=[jax.ShapeDtypeStruct((_NPAD, _D), jnp.float32),
                   jax.ShapeDtypeStruct((_NPAD, _DW), jnp.float32)],
    )(d0, d1, xp, wb, wc, bc)


def _tc_mid(s0, s1, bd, w, d0, d1, bias, wb_n, wc_n, bc_n):
    return pl.pallas_call(
        _tc_mid_body,
        grid=(_GRID,),
        in_specs=[_bn(_D), _bn(_D), _bn(_D), _bn(_DW), _bn(16), _bn(16),
                  _full(1, 128), _full(128, _D), _full(128, _DW), _full(1, _DW)],
        out_specs=[_bn(_D), _bn(_DW)],
        out_shape=[jax.ShapeDtypeStruct((_NPAD, _D), jnp.float32),
                   jax.ShapeDtypeStruct((_NPAD, _DW), jnp.float32)],
    )(s0, s1, bd, w, d0, d1, bias, wb_n, wc_n, bc_n)


def _tc_last(s0, s1, bd, w, d0, d1, bias):
    return pl.pallas_call(
        _tc_last_body,
        grid=(_GRID,),
        in_specs=[_bn(_D), _bn(_D), _bn(_D), _bn(_DW), _bn(16), _bn(16),
                  _full(1, 128)],
        out_specs=_bn(128),
        out_shape=jax.ShapeDtypeStruct((_NPAD, 128), jnp.float32),
    )(s0, s1, bd, w, d0, d1, bias)


_PERM = np.array([(j % _HEADS) * _NB + j // _HEADS for j in range(_DW)])


def kernel(x, edge_index, edge_attr,
           Wb1, Wc1, bc1, b1, Wb2, Wc2, bc2, b2,
           Wb3, Wc3, bc3, b3, Wb4, Wc4, bc4, b4):
    f32 = jnp.float32
    xp = jnp.pad(x, ((0, _NPAD - _N), (0, 0)))
    pad = _EP - _E
    row = jnp.concatenate(
        [edge_index[0], jnp.zeros((pad,), jnp.int32)]).reshape(_NW, _NCHUNK, _CH)
    col = jnp.concatenate(
        [edge_index[1], jnp.full((pad,), _N, jnp.int32)]).reshape(_NW, _NCHUNK, _CH)
    ones16 = jnp.ones((_CH, 16), f32)
    z16 = jnp.zeros((_RPT, 16), f32)
    z64 = jnp.zeros((_RPT, _D), f32)

    wbs = [Wb1, Wb2, Wb3, Wb4]
    wcs = [Wc1[:, _PERM], Wc2[:, _PERM], Wc3[:, _PERM], Wc4[:, _PERM]]
    bcs = [bc1[_PERM].reshape(1, _DW), bc2[_PERM].reshape(1, _DW),
           bc3[_PERM].reshape(1, _DW), bc4[_PERM].reshape(1, _DW)]
    bs = [b1.reshape(1, 128), b2.reshape(1, 128),
          b3.reshape(1, 128), b4.reshape(1, 128)]

    degp = _sc_deg()(col, ones16, z16)
    d0, d1 = degp[:_NPAD], degp[_NPAD:]

    gs = _sc_gather_scatter()
    bd, w = _tc_first(d0, d1, xp, wbs[0], wcs[0], bcs[0])
    for l in range(3):
        s = gs(bd, row, col, z64)
        bd, w = _tc_mid(s[:_NPAD], s[_NPAD:], bd, w, d0, d1,
                        bs[l], wbs[l + 1], wcs[l + 1], bcs[l + 1])
    s = gs(bd, row, col, z64)
    out = _tc_last(s[:_NPAD], s[_NPAD:], bd, w, d0, d1, bs[3])
    return out[:_N]


# speedup vs baseline: 12.5177x; 1.0005x over previous
import functools

import numpy as np
import jax
import jax.numpy as jnp
from jax import lax
from jax.experimental import pallas as pl
from jax.experimental.pallas import tpu as pltpu
from jax.experimental.pallas import tpu_sc as plsc

_N = 10000
_NPAD = 10240
_E = 320000
_NW = 32
_CH = 128
_NCHUNK = 80
_EP = _NW * _NCHUNK * _CH
_D = 64
_DW = 32
_RPT = _NPAD // 16
_BN = 512
_GRID = _NPAD // _BN
_HEADS, _NB, _HD = 8, 4, 16
_NBUF = 4


@functools.lru_cache(maxsize=None)
def _sc_deg():
    mesh = plsc.VectorSubcoreMesh(core_axis_name="c", subcore_axis_name="s")

    @functools.partial(
        pl.kernel, mesh=mesh,
        compiler_params=pltpu.CompilerParams(use_tc_tiling_on_sc=False),
        out_type=jax.ShapeDtypeStruct((2 * _NPAD, 16), jnp.float32),
        scratch_types=[
            pltpu.VMEM((_NCHUNK, _CH), jnp.int32),
            pltpu.VMEM((_CH, 16), jnp.float32),
            pltpu.VMEM_SHARED((_NPAD, 16), jnp.float32),
        ],
    )
    def deg_kernel(coli, ones, z16, out, colv, onesv, acc):
        c = lax.axis_index("c")
        s = lax.axis_index("s")
        wid = s * 2 + c
        pltpu.sync_copy(coli.at[wid], colv)
        pltpu.sync_copy(ones, onesv)
        pltpu.sync_copy(z16, acc.at[pl.ds(s * _RPT, _RPT)])
        plsc.subcore_barrier()

        def body(j, carry):
            pltpu.sync_copy(onesv, acc.at[colv.at[j]], add=True)
            return carry

        lax.fori_loop(0, _NCHUNK, body, 0)
        plsc.subcore_barrier()
        pltpu.sync_copy(acc.at[pl.ds(s * _RPT, _RPT)],
                        out.at[pl.ds(c * _NPAD + s * _RPT, _RPT)])

    return deg_kernel


@functools.lru_cache(maxsize=None)
def _sc_gather_scatter():
    mesh = plsc.VectorSubcoreMesh(core_axis_name="c", subcore_axis_name="s")

    @functools.partial(
        pl.kernel, mesh=mesh,
        compiler_params=pltpu.CompilerParams(use_tc_tiling_on_sc=False),
        out_type=jax.ShapeDtypeStruct((2 * _NPAD, _D), jnp.float32),
        scratch_types=[
            pltpu.VMEM((_NCHUNK, _CH), jnp.int32),
            pltpu.VMEM((_NCHUNK, _CH), jnp.int32),
            [pltpu.VMEM((_CH, _D), jnp.float32) for _ in range(_NBUF)],
            pltpu.VMEM_SHARED((_NPAD, _D), jnp.float32),
            [pltpu.SemaphoreType.DMA for _ in range(_NBUF)],
            [pltpu.SemaphoreType.DMA for _ in range(_NBUF)],
        ],
    )
    def gs_kernel(table, rowi, coli, z64, out,
                  rowv, colv, bufs, acc, gsems, ssems):
        c = lax.axis_index("c")
        s = lax.axis_index("s")
        wid = s * 2 + c
        pltpu.sync_copy(rowi.at[wid], rowv)
        pltpu.sync_copy(coli.at[wid], colv)
        pltpu.sync_copy(z64, acc.at[pl.ds(s * _RPT, _RPT)])
        plsc.subcore_barrier()

        for b in range(_NBUF):
            pltpu.async_copy(table.at[rowv.at[b]], bufs[b], gsems[b])

        def body(jj, carry):
            j = _NBUF * jj
            for b in range(_NBUF):
                pltpu.make_async_copy(
                    table.at[rowv.at[j + b]], bufs[b], gsems[b]).wait()
                pltpu.async_copy(bufs[b], acc.at[colv.at[j + b]], ssems[b],
                                 add=True)
            @pl.when(jj < _NCHUNK // _NBUF - 1)
            def _():
                for b in range(_NBUF):
                    pltpu.make_async_copy(
                        bufs[b], acc.at[colv.at[j + b]], ssems[b]).wait()
                    pltpu.async_copy(
                        table.at[rowv.at[j + _NBUF + b]], bufs[b], gsems[b])
            return carry

        lax.fori_loop(0, _NCHUNK // _NBUF, body, 0)
        j_last = _NCHUNK - _NBUF
        for b in range(_NBUF):
            pltpu.make_async_copy(
                bufs[b], acc.at[colv.at[j_last + b]], ssems[b]).wait()
        plsc.subcore_barrier()
        pltpu.sync_copy(acc.at[pl.ds(s * _RPT, _RPT)],
                        out.at[pl.ds(c * _NPAD + s * _RPT, _RPT)])

    return gs_kernel


def _expm1(v):
    u = jnp.exp(v)
    return jnp.where(u == 1.0, v,
                     jnp.where(u > 0.0, (u - 1.0) * v / jnp.log(u), -1.0))


def _elu1(v):
    return jnp.where(v > 0, v, _expm1(v))


def _combine(w, agg, bias):
    r8 = lax.broadcasted_iota(jnp.int32, (_HEADS, 128), 0)
    c8 = lax.broadcasted_iota(jnp.int32, (_HEADS, 128), 1)
    e8 = (c8 // _HD == r8).astype(jnp.float32)
    r16 = lax.broadcasted_iota(jnp.int32, (_HD, 128), 0)
    c16 = lax.broadcasted_iota(jnp.int32, (_HD, 128), 1)
    e16 = (c16 % _HD == r16).astype(jnp.float32)
    out = bias
    for b in range(_NB):
        rep = jnp.dot(w[:, b * _HEADS:(b + 1) * _HEADS], e8,
                      preferred_element_type=jnp.float32)
        til = jnp.dot(agg[:, b * _HD:(b + 1) * _HD], e16,
                      preferred_element_type=jnp.float32)
        out = out + rep * til
    return out


def _dinv_of(d0, d1):
    return lax.rsqrt(1.0 + d0[...][:, :1] + d1[...][:, :1])


def _tc_first_body(d0, d1, x, wb, wc, bc, ob, ow):
    dinv = _dinv_of(d0, d1)
    xb = x[...]
    ob[...] = dinv * jnp.dot(xb, wb[...], preferred_element_type=jnp.float32)
    ow[...] = jnp.dot(xb, wc[...], preferred_element_type=jnp.float32) + bc[...]


def _tc_mid_body(s0, s1, bd, w, d0, d1, bias, wb_n, wc_n, bc_n, ob, ow):
    dinv = _dinv_of(d0, d1)
    agg = dinv * (s0[...] + s1[...] + bd[...])
    h = _elu1(_combine(w[...], agg, bias[...]))
    ob[...] = dinv * jnp.dot(h, wb_n[...], preferred_element_type=jnp.float32)
    ow[...] = jnp.dot(h, wc_n[...], preferred_element_type=jnp.float32) + bc_n[...]


def _tc_last_body(s0, s1, bd, w, d0, d1, bias, o):
    dinv = _dinv_of(d0, d1)
    agg = dinv * (s0[...] + s1[...] + bd[...])
    v = _combine(w[...], agg, bias[...])
    o[...] = jnp.where(v > 0, v, 128.0 * _expm1(v))


def _bn(width):
    return pl.BlockSpec((_BN, width), lambda i: (i, 0))


def _bn_hi(width):
    return pl.BlockSpec((_BN, width), lambda i: (i + _GRID, 0))


def _full(rows, cols):
    return pl.BlockSpec((rows, cols), lambda i: (0, 0))


def _tc_first(degp, xp, wb, wc, bc):
    return pl.pallas_call(
        _tc_first_body,
        grid=(_GRID,),
        in_specs=[_bn(16), _bn_hi(16), _bn(128),
                  _full(128, _D), _full(128, _DW), _full(1, _DW)],
        out_specs=[_bn(_D), _bn(_DW)],
        out_shape=[jax.ShapeDtypeStruct((_NPAD, _D), jnp.float32),
                   jax.ShapeDtypeStruct((_NPAD, _DW), jnp.float32)],
    )(degp, degp, xp, wb, wc, bc)


def _tc_mid(s, bd, w, degp, bias, wb_n, wc_n, bc_n):
    return pl.pallas_call(
        _tc_mid_body,
        grid=(_GRID,),
        in_specs=[_bn(_D), _bn_hi(_D), _bn(_D), _bn(_DW), _bn(16), _bn_hi(16),
                  _full(1, 128), _full(128, _D), _full(128, _DW), _full(1, _DW)],
        out_specs=[_bn(_D), _bn(_DW)],
        out_shape=[jax.ShapeDtypeStruct((_NPAD, _D), jnp.float32),
                   jax.ShapeDtypeStruct((_NPAD, _DW), jnp.float32)],
    )(s, s, bd, w, degp, degp, bias, wb_n, wc_n, bc_n)


def _tc_last(s, bd, w, degp, bias):
    return pl.pallas_call(
        _tc_last_body,
        grid=(_GRID,),
        in_specs=[_bn(_D), _bn_hi(_D), _bn(_D), _bn(_DW), _bn(16), _bn_hi(16),
                  _full(1, 128)],
        out_specs=_bn(128),
        out_shape=jax.ShapeDtypeStruct((_NPAD, 128), jnp.float32),
    )(s, s, bd, w, degp, degp, bias)


_PERM = np.array([(j % _HEADS) * _NB + j // _HEADS for j in range(_DW)])


def kernel(x, edge_index, edge_attr,
           Wb1, Wc1, bc1, b1, Wb2, Wc2, bc2, b2,
           Wb3, Wc3, bc3, b3, Wb4, Wc4, bc4, b4):
    f32 = jnp.float32
    xp = jnp.pad(x, ((0, _NPAD - _N), (0, 0)))
    pad = _EP - _E
    row = jnp.concatenate(
        [edge_index[0], jnp.zeros((pad,), jnp.int32)]).reshape(_NW, _NCHUNK, _CH)
    col = jnp.concatenate(
        [edge_index[1], jnp.full((pad,), _N, jnp.int32)]).reshape(_NW, _NCHUNK, _CH)
    ones16 = jnp.ones((_CH, 16), f32)
    z16 = jnp.zeros((_RPT, 16), f32)
    z64 = jnp.zeros((_RPT, _D), f32)

    wbs = [Wb1, Wb2, Wb3, Wb4]
    wcs = [Wc1[:, _PERM], Wc2[:, _PERM], Wc3[:, _PERM], Wc4[:, _PERM]]
    bcs = [bc1[_PERM].reshape(1, _DW), bc2[_PERM].reshape(1, _DW),
           bc3[_PERM].reshape(1, _DW), bc4[_PERM].reshape(1, _DW)]
    bs = [b1.reshape(1, 128), b2.reshape(1, 128),
          b3.reshape(1, 128), b4.reshape(1, 128)]

    degp = _sc_deg()(col, ones16, z16)

    gs = _sc_gather_scatter()
    bd, w = _tc_first(degp, xp, wbs[0], wcs[0], bcs[0])
    for l in range(3):
        s = gs(bd, row, col, z64)
        bd, w = _tc_mid(s, bd, w, degp,
                        bs[l], wbs[l + 1], wcs[l + 1], bcs[l + 1])
    s = gs(bd, row, col, z64)
    out = _tc_last(s, bd, w, degp, bs[3])
    return out[:_N]


# speedup vs baseline: 25.3395x; 2.0243x over previous
import functools

import numpy as np
import jax
import jax.numpy as jnp
from jax import lax
from jax.experimental import pallas as pl
from jax.experimental.pallas import tpu as pltpu
from jax.experimental.pallas import tpu_sc as plsc

_N = 10000
_NPAD = 10240
_E = 320000
_NW = 32
_CH = 128
_NCHUNK = 80
_EP = _NW * _NCHUNK * _CH
_D = 64
_DW = 32
_RPT = _NPAD // 16
_BN = 512
_GRID = _NPAD // _BN
_HEADS, _NB, _HD = 8, 4, 16
_NBUF = 2


@functools.lru_cache(maxsize=None)
def _sc_deg():
    mesh = plsc.VectorSubcoreMesh(core_axis_name="c", subcore_axis_name="s")

    @functools.partial(
        pl.kernel, mesh=mesh,
        compiler_params=pltpu.CompilerParams(use_tc_tiling_on_sc=False),
        out_type=jax.ShapeDtypeStruct((2 * _NPAD, 16), jnp.float32),
        scratch_types=[
            pltpu.VMEM((_NCHUNK, _CH), jnp.int32),
            pltpu.VMEM((_NCHUNK, _CH), jnp.int32),
            pltpu.VMEM((_CH, 16), jnp.float32),
            pltpu.VMEM_SHARED((_NPAD, 16), jnp.float32),
        ],
    )
    def deg_kernel(pk, ones, z16, out, pv, colv, onesv, acc):
        c = lax.axis_index("c")
        s = lax.axis_index("s")
        wid = s * 2 + c
        pltpu.sync_copy(pk.at[wid], pv)
        pltpu.sync_copy(ones, onesv)
        pltpu.sync_copy(z16, acc.at[pl.ds(s * _RPT, _RPT)])

        def unp(j, carry):
            for k in range(_CH // 16):
                v = pv[j, pl.ds(k * 16, 16)]
                colv[j, pl.ds(k * 16, 16)] = lax.shift_right_logical(v, 14)
            return carry

        lax.fori_loop(0, _NCHUNK, unp, 0)
        plsc.subcore_barrier()

        def body(j, carry):
            pltpu.sync_copy(onesv, acc.at[colv.at[j]], add=True)
            return carry

        lax.fori_loop(0, _NCHUNK, body, 0)
        plsc.subcore_barrier()
        pltpu.sync_copy(acc.at[pl.ds(s * _RPT, _RPT)],
                        out.at[pl.ds(c * _NPAD + s * _RPT, _RPT)])

    return deg_kernel


@functools.lru_cache(maxsize=None)
def _sc_gather_scatter():
    mesh = plsc.VectorSubcoreMesh(core_axis_name="c", subcore_axis_name="s")

    @functools.partial(
        pl.kernel, mesh=mesh,
        compiler_params=pltpu.CompilerParams(use_tc_tiling_on_sc=False),
        out_type=jax.ShapeDtypeStruct((2 * _NPAD, _D), jnp.float32),
        scratch_types=[
            pltpu.VMEM((_NCHUNK, _CH), jnp.int32),
            pltpu.VMEM((_NCHUNK, _CH), jnp.int32),
            [pltpu.VMEM((_CH, _D), jnp.float32) for _ in range(_NBUF)],
            pltpu.VMEM_SHARED((_NPAD, _D), jnp.float32),
            pltpu.VMEM_SHARED((_NPAD, _D), jnp.float32),
            [pltpu.SemaphoreType.DMA for _ in range(_NBUF)],
            [pltpu.SemaphoreType.DMA for _ in range(_NBUF)],
        ],
    )
    def gs_kernel(table, pk, z64, out,
                  rowv, colv, bufs, acc, tsp, gsems, ssems):
        c = lax.axis_index("c")
        s = lax.axis_index("s")
        wid = s * 2 + c
        pltpu.sync_copy(pk.at[wid], rowv)
        pltpu.sync_copy(z64, acc.at[pl.ds(s * _RPT, _RPT)])
        pltpu.sync_copy(table.at[pl.ds(s * _RPT, _RPT)],
                        tsp.at[pl.ds(s * _RPT, _RPT)])

        def unp(j, carry):
            for k in range(_CH // 16):
                v = rowv[j, pl.ds(k * 16, 16)]
                colv[j, pl.ds(k * 16, 16)] = lax.shift_right_logical(v, 14)
                rowv[j, pl.ds(k * 16, 16)] = v & 16383
            return carry

        lax.fori_loop(0, _NCHUNK, unp, 0)
        plsc.subcore_barrier()

        for b in range(_NBUF):
            pltpu.async_copy(tsp.at[rowv.at[b]], bufs[b], gsems[b])

        def body(jj, carry):
            j = _NBUF * jj
            for b in range(_NBUF):
                pltpu.make_async_copy(
                    tsp.at[rowv.at[j + b]], bufs[b], gsems[b]).wait()
                pltpu.async_copy(bufs[b], acc.at[colv.at[j + b]], ssems[b],
                                 add=True)
            @pl.when(jj < _NCHUNK // _NBUF - 1)
            def _():
                for b in range(_NBUF):
                    pltpu.make_async_copy(
                        bufs[b], acc.at[colv.at[j + b]], ssems[b]).wait()
                    pltpu.async_copy(
                        tsp.at[rowv.at[j + _NBUF + b]], bufs[b], gsems[b])
            return carry

        lax.fori_loop(0, _NCHUNK // _NBUF, body, 0)
        j_last = _NCHUNK - _NBUF
        for b in range(_NBUF):
            pltpu.make_async_copy(
                bufs[b], acc.at[colv.at[j_last + b]], ssems[b]).wait()
        plsc.subcore_barrier()
        pltpu.sync_copy(acc.at[pl.ds(s * _RPT, _RPT)],
                        out.at[pl.ds(c * _NPAD + s * _RPT, _RPT)])

    return gs_kernel


def _expm1(v):
    u = jnp.exp(v)
    return jnp.where(u == 1.0, v,
                     jnp.where(u > 0.0, (u - 1.0) * v / jnp.log(u), -1.0))


def _elu1(v):
    return jnp.where(v > 0, v, _expm1(v))


def _combine(w, agg, bias):
    r8 = lax.broadcasted_iota(jnp.int32, (_HEADS, 128), 0)
    c8 = lax.broadcasted_iota(jnp.int32, (_HEADS, 128), 1)
    e8 = (c8 // _HD == r8).astype(jnp.float32)
    r16 = lax.broadcasted_iota(jnp.int32, (_HD, 128), 0)
    c16 = lax.broadcasted_iota(jnp.int32, (_HD, 128), 1)
    e16 = (c16 % _HD == r16).astype(jnp.float32)
    out = bias
    for b in range(_NB):
        rep = jnp.dot(w[:, b * _HEADS:(b + 1) * _HEADS], e8,
                      preferred_element_type=jnp.float32)
        til = jnp.dot(agg[:, b * _HD:(b + 1) * _HD], e16,
                      preferred_element_type=jnp.float32)
        out = out + rep * til
    return out


def _dinv_of(d0, d1):
    return lax.rsqrt(1.0 + d0[...][:, :1] + d1[...][:, :1])


def _tc_first_body(d0, d1, x, wb, wc, bc, ob, ow):
    dinv = _dinv_of(d0, d1)
    xb = x[...]
    ob[...] = dinv * jnp.dot(xb, wb[...], preferred_element_type=jnp.float32)
    ow[...] = jnp.dot(xb, wc[...], preferred_element_type=jnp.float32) + bc[...]


def _tc_mid_body(s0, s1, bd, w, d0, d1, bias, wb_n, wc_n, bc_n, ob, ow):
    dinv = _dinv_of(d0, d1)
    agg = dinv * (s0[...] + s1[...] + bd[...])
    h = _elu1(_combine(w[...], agg, bias[...]))
    ob[...] = dinv * jnp.dot(h, wb_n[...], preferred_element_type=jnp.float32)
    ow[...] = jnp.dot(h, wc_n[...], preferred_element_type=jnp.float32) + bc_n[...]


def _tc_last_body(s0, s1, bd, w, d0, d1, bias, o):
    dinv = _dinv_of(d0, d1)
    agg = dinv * (s0[...] + s1[...] + bd[...])
    v = _combine(w[...], agg, bias[...])
    o[...] = jnp.where(v > 0, v, 128.0 * _expm1(v))


def _bn(width):
    return pl.BlockSpec((_BN, width), lambda i: (i, 0))


def _bn_hi(width):
    return pl.BlockSpec((_BN, width), lambda i: (i + _GRID, 0))


def _full(rows, cols):
    return pl.BlockSpec((rows, cols), lambda i: (0, 0))


def _tc_first(degp, xp, wb, wc, bc):
    return pl.pallas_call(
        _tc_first_body,
        grid=(_GRID,),
        in_specs=[_bn(16), _bn_hi(16), _bn(128),
                  _full(128, _D), _full(128, _DW), _full(1, _DW)],
        out_specs=[_bn(_D), _bn(_DW)],
        out_shape=[jax.ShapeDtypeStruct((_NPAD, _D), jnp.float32),
                   jax.ShapeDtypeStruct((_NPAD, _DW), jnp.float32)],
    )(degp, degp, xp, wb, wc, bc)


def _tc_mid(s, bd, w, degp, bias, wb_n, wc_n, bc_n):
    return pl.pallas_call(
        _tc_mid_body,
        grid=(_GRID,),
        in_specs=[_bn(_D), _bn_hi(_D), _bn(_D), _bn(_DW), _bn(16), _bn_hi(16),
                  _full(1, 128), _full(128, _D), _full(128, _DW), _full(1, _DW)],
        out_specs=[_bn(_D), _bn(_DW)],
        out_shape=[jax.ShapeDtypeStruct((_NPAD, _D), jnp.float32),
                   jax.ShapeDtypeStruct((_NPAD, _DW), jnp.float32)],
    )(s, s, bd, w, degp, degp, bias, wb_n, wc_n, bc_n)


def _tc_last(s, bd, w, degp, bias):
    return pl.pallas_call(
        _tc_last_body,
        grid=(_GRID,),
        in_specs=[_bn(_D), _bn_hi(_D), _bn(_D), _bn(_DW), _bn(16), _bn_hi(16),
                  _full(1, 128)],
        out_specs=_bn(128),
        out_shape=jax.ShapeDtypeStruct((_NPAD, 128), jnp.float32),
    )(s, s, bd, w, degp, degp, bias)


_PERM = np.array([(j % _HEADS) * _NB + j // _HEADS for j in range(_DW)])


def kernel(x, edge_index, edge_attr,
           Wb1, Wc1, bc1, b1, Wb2, Wc2, bc2, b2,
           Wb3, Wc3, bc3, b3, Wb4, Wc4, bc4, b4):
    f32 = jnp.float32
    xp = jnp.pad(x, ((0, _NPAD - _N), (0, 0)))
    pad = _EP - _E
    pk = jnp.concatenate(
        [edge_index[0] | (edge_index[1] << 14),
         jnp.full((pad,), _N << 14, jnp.int32)]).reshape(_NW, _NCHUNK, _CH)
    ones16 = jnp.ones((_CH, 16), f32)
    z16 = jnp.zeros((_RPT, 16), f32)
    z64 = jnp.zeros((_RPT, _D), f32)

    wbs = [Wb1, Wb2, Wb3, Wb4]
    wcs = [Wc1[:, _PERM], Wc2[:, _PERM], Wc3[:, _PERM], Wc4[:, _PERM]]
    bcs = [bc1[_PERM].reshape(1, _DW), bc2[_PERM].reshape(1, _DW),
           bc3[_PERM].reshape(1, _DW), bc4[_PERM].reshape(1, _DW)]
    bs = [b1.reshape(1, 128), b2.reshape(1, 128),
          b3.reshape(1, 128), b4.reshape(1, 128)]

    degp = _sc_deg()(pk, ones16, z16)

    gs = _sc_gather_scatter()
    bd, w = _tc_first(degp, xp, wbs[0], wcs[0], bcs[0])
    for l in range(3):
        s = gs(bd, pk, z64)
        bd, w = _tc_mid(s, bd, w, degp,
                        bs[l], wbs[l + 1], wcs[l + 1], bcs[l + 1])
    s = gs(bd, pk, z64)
    out = _tc_last(s, bd, w, degp, bs[3])
    return out[:_N]


# speedup vs baseline: 25.4769x; 1.0054x over previous
import functools

import numpy as np
import jax
import jax.numpy as jnp
from jax import lax
from jax.experimental import pallas as pl
from jax.experimental.pallas import tpu as pltpu
from jax.experimental.pallas import tpu_sc as plsc

_N = 10000
_NPAD = 10240
_E = 320000
_NW = 32
_CH = 128
_NCHUNK = 80
_EP = _NW * _NCHUNK * _CH
_D = 64
_DW = 32
_RPT = _NPAD // 16
_BN = 512
_GRID = _NPAD // _BN
_HEADS, _NB, _HD = 8, 4, 16
_NBUF = 2


@functools.lru_cache(maxsize=None)
def _sc_deg():
    mesh = plsc.VectorSubcoreMesh(core_axis_name="c", subcore_axis_name="s")

    @functools.partial(
        pl.kernel, mesh=mesh,
        compiler_params=pltpu.CompilerParams(use_tc_tiling_on_sc=False),
        out_type=jax.ShapeDtypeStruct((2 * _NPAD, 16), jnp.float32),
        scratch_types=[
            pltpu.VMEM((_NCHUNK, _CH), jnp.int32),
            pltpu.VMEM((_NCHUNK, _CH), jnp.int32),
            pltpu.VMEM((_CH, 16), jnp.float32),
            pltpu.VMEM_SHARED((_NPAD, 16), jnp.float32),
        ],
    )
    def deg_kernel(pk, ones, z16, out, pv, colv, onesv, acc):
        c = lax.axis_index("c")
        s = lax.axis_index("s")
        wid = s * 2 + c
        pltpu.sync_copy(pk.at[wid], pv)
        pltpu.sync_copy(ones, onesv)
        pltpu.sync_copy(z16, acc.at[pl.ds(s * _RPT, _RPT)])

        def unp(j, carry):
            for k in range(_CH // 16):
                v = pv[j, pl.ds(k * 16, 16)]
                colv[j, pl.ds(k * 16, 16)] = lax.shift_right_logical(v, 14)
            return carry

        lax.fori_loop(0, _NCHUNK, unp, 0)
        plsc.subcore_barrier()

        def body(j, carry):
            pltpu.sync_copy(onesv, acc.at[colv.at[j]], add=True)
            return carry

        lax.fori_loop(0, _NCHUNK, body, 0)
        plsc.subcore_barrier()
        pltpu.sync_copy(acc.at[pl.ds(s * _RPT, _RPT)],
                        out.at[pl.ds(c * _NPAD + s * _RPT, _RPT)])

    return deg_kernel


@functools.lru_cache(maxsize=None)
def _sc_gather_scatter():
    mesh = plsc.VectorSubcoreMesh(core_axis_name="c", subcore_axis_name="s")

    @functools.partial(
        pl.kernel, mesh=mesh,
        compiler_params=pltpu.CompilerParams(use_tc_tiling_on_sc=False),
        out_type=jax.ShapeDtypeStruct((2 * _NPAD, _D), jnp.float32),
        scratch_types=[
            pltpu.VMEM((_NCHUNK, _CH), jnp.int32),
            pltpu.VMEM((_NCHUNK, _CH), jnp.int32),
            [pltpu.VMEM((_CH, _D), jnp.float32) for _ in range(_NBUF)],
            pltpu.VMEM_SHARED((_NPAD, _D), jnp.float32),
            pltpu.VMEM_SHARED((_NPAD, _D), jnp.float32),
            [pltpu.SemaphoreType.DMA for _ in range(_NBUF)],
            [pltpu.SemaphoreType.DMA for _ in range(_NBUF)],
        ],
    )
    def gs_kernel(table, pk, z64, out,
                  rowv, colv, bufs, acc, tsp, gsems, ssems):
        c = lax.axis_index("c")
        s = lax.axis_index("s")
        wid = s * 2 + c
        pltpu.sync_copy(pk.at[wid], rowv)
        @pl.when(c == 0)
        def _():
            pltpu.sync_copy(table.at[pl.ds(s * _RPT, _RPT)],
                            acc.at[pl.ds(s * _RPT, _RPT)])

        @pl.when(c != 0)
        def _():
            pltpu.sync_copy(z64, acc.at[pl.ds(s * _RPT, _RPT)])
        pltpu.sync_copy(table.at[pl.ds(s * _RPT, _RPT)],
                        tsp.at[pl.ds(s * _RPT, _RPT)])

        def unp(j, carry):
            for k in range(_CH // 16):
                v = rowv[j, pl.ds(k * 16, 16)]
                colv[j, pl.ds(k * 16, 16)] = lax.shift_right_logical(v, 14)
                rowv[j, pl.ds(k * 16, 16)] = v & 16383
            return carry

        lax.fori_loop(0, _NCHUNK, unp, 0)
        plsc.subcore_barrier()

        for b in range(_NBUF):
            pltpu.async_copy(tsp.at[rowv.at[b]], bufs[b], gsems[b])

        def body(jj, carry):
            j = _NBUF * jj
            for b in range(_NBUF):
                pltpu.make_async_copy(
                    tsp.at[rowv.at[j + b]], bufs[b], gsems[b]).wait()
                pltpu.async_copy(bufs[b], acc.at[colv.at[j + b]], ssems[b],
                                 add=True)
            @pl.when(jj < _NCHUNK // _NBUF - 1)
            def _():
                for b in range(_NBUF):
                    pltpu.make_async_copy(
                        bufs[b], acc.at[colv.at[j + b]], ssems[b]).wait()
                    pltpu.async_copy(
                        tsp.at[rowv.at[j + _NBUF + b]], bufs[b], gsems[b])
            return carry

        lax.fori_loop(0, _NCHUNK // _NBUF, body, 0)
        j_last = _NCHUNK - _NBUF
        for b in range(_NBUF):
            pltpu.make_async_copy(
                bufs[b], acc.at[colv.at[j_last + b]], ssems[b]).wait()
        plsc.subcore_barrier()
        pltpu.sync_copy(acc.at[pl.ds(s * _RPT, _RPT)],
                        out.at[pl.ds(c * _NPAD + s * _RPT, _RPT)])

    return gs_kernel


def _expm1(v):
    u = jnp.exp(v)
    return jnp.where(u == 1.0, v,
                     jnp.where(u > 0.0, (u - 1.0) * v / jnp.log(u), -1.0))


def _elu1(v):
    return jnp.where(v > 0, v, _expm1(v))


def _combine(w, agg, bias):
    r8 = lax.broadcasted_iota(jnp.int32, (_HEADS, 128), 0)
    c8 = lax.broadcasted_iota(jnp.int32, (_HEADS, 128), 1)
    e8 = (c8 // _HD == r8).astype(jnp.float32)
    r16 = lax.broadcasted_iota(jnp.int32, (_HD, 128), 0)
    c16 = lax.broadcasted_iota(jnp.int32, (_HD, 128), 1)
    e16 = (c16 % _HD == r16).astype(jnp.float32)
    out = bias
    for b in range(_NB):
        rep = jnp.dot(w[:, b * _HEADS:(b + 1) * _HEADS], e8,
                      preferred_element_type=jnp.float32)
        til = jnp.dot(agg[:, b * _HD:(b + 1) * _HD], e16,
                      preferred_element_type=jnp.float32)
        out = out + rep * til
    return out


def _dinv_of(d0, d1):
    return lax.rsqrt(1.0 + d0[...][:, :1] + d1[...][:, :1])


def _tc_first_body(d0, d1, x, wb, wc, bc, ob, ow):
    dinv = _dinv_of(d0, d1)
    xb = x[...]
    ob[...] = dinv * jnp.dot(xb, wb[...], preferred_element_type=jnp.float32)
    ow[...] = jnp.dot(xb, wc[...], preferred_element_type=jnp.float32) + bc[...]


def _tc_mid_body(s0, s1, w, d0, d1, bias, wb_n, wc_n, bc_n, ob, ow):
    dinv = _dinv_of(d0, d1)
    agg = dinv * (s0[...] + s1[...])
    h = _elu1(_combine(w[...], agg, bias[...]))
    ob[...] = dinv * jnp.dot(h, wb_n[...], preferred_element_type=jnp.float32)
    ow[...] = jnp.dot(h, wc_n[...], preferred_element_type=jnp.float32) + bc_n[...]


def _tc_last_body(s0, s1, w, d0, d1, bias, o):
    dinv = _dinv_of(d0, d1)
    agg = dinv * (s0[...] + s1[...])
    v = _combine(w[...], agg, bias[...])
    o[...] = jnp.where(v > 0, v, 128.0 * _expm1(v))


def _bn(width):
    return pl.BlockSpec((_BN, width), lambda i: (i, 0))


def _bn_hi(width):
    return pl.BlockSpec((_BN, width), lambda i: (i + _GRID, 0))


def _full(rows, cols):
    return pl.BlockSpec((rows, cols), lambda i: (0, 0))


def _tc_first(degp, xp, wb, wc, bc):
    return pl.pallas_call(
        _tc_first_body,
        grid=(_GRID,),
        in_specs=[_bn(16), _bn_hi(16), _bn(128),
                  _full(128, _D), _full(128, _DW), _full(1, _DW)],
        out_specs=[_bn(_D), _bn(_DW)],
        out_shape=[jax.ShapeDtypeStruct((_NPAD, _D), jnp.float32),
                   jax.ShapeDtypeStruct((_NPAD, _DW), jnp.float32)],
    )(degp, degp, xp, wb, wc, bc)


def _tc_mid(s, w, degp, bias, wb_n, wc_n, bc_n):
    return pl.pallas_call(
        _tc_mid_body,
        grid=(_GRID,),
        in_specs=[_bn(_D), _bn_hi(_D), _bn(_DW), _bn(16), _bn_hi(16),
                  _full(1, 128), _full(128, _D), _full(128, _DW), _full(1, _DW)],
        out_specs=[_bn(_D), _bn(_DW)],
        out_shape=[jax.ShapeDtypeStruct((_NPAD, _D), jnp.float32),
                   jax.ShapeDtypeStruct((_NPAD, _DW), jnp.float32)],
    )(s, s, w, degp, degp, bias, wb_n, wc_n, bc_n)


def _tc_last(s, w, degp, bias):
    return pl.pallas_call(
        _tc_last_body,
        grid=(_GRID,),
        in_specs=[_bn(_D), _bn_hi(_D), _bn(_DW), _bn(16), _bn_hi(16),
                  _full(1, 128)],
        out_specs=_bn(128),
        out_shape=jax.ShapeDtypeStruct((_NPAD, 128), jnp.float32),
    )(s, s, w, degp, degp, bias)


_PERM = np.array([(j % _HEADS) * _NB + j // _HEADS for j in range(_DW)])


def kernel(x, edge_index, edge_attr,
           Wb1, Wc1, bc1, b1, Wb2, Wc2, bc2, b2,
           Wb3, Wc3, bc3, b3, Wb4, Wc4, bc4, b4):
    f32 = jnp.float32
    xp = jnp.pad(x, ((0, _NPAD - _N), (0, 0)))
    pad = _EP - _E
    pk = jnp.concatenate(
        [edge_index[0] | (edge_index[1] << 14),
         jnp.full((pad,), _N << 14, jnp.int32)]).reshape(_NW, _NCHUNK, _CH)
    ones16 = jnp.ones((_CH, 16), f32)
    z16 = jnp.zeros((_RPT, 16), f32)
    z64 = jnp.zeros((_RPT, _D), f32)

    wbs = [Wb1, Wb2, Wb3, Wb4]
    wcs = [Wc1[:, _PERM], Wc2[:, _PERM], Wc3[:, _PERM], Wc4[:, _PERM]]
    bcs = [bc1[_PERM].reshape(1, _DW), bc2[_PERM].reshape(1, _DW),
           bc3[_PERM].reshape(1, _DW), bc4[_PERM].reshape(1, _DW)]
    bs = [b1.reshape(1, 128), b2.reshape(1, 128),
          b3.reshape(1, 128), b4.reshape(1, 128)]

    degp = _sc_deg()(pk, ones16, z16)

    gs = _sc_gather_scatter()
    bd, w = _tc_first(degp, xp, wbs[0], wcs[0], bcs[0])
    for l in range(3):
        s = gs(bd, pk, z64)
        bd, w = _tc_mid(s, w, degp,
                        bs[l], wbs[l + 1], wcs[l + 1], bcs[l + 1])
    s = gs(bd, pk, z64)
    out = _tc_last(s, w, degp, bs[3])
    return out[:_N]
